# baseline (speedup 1.0000x reference)
import jax
import jax.numpy as jnp
from jax.experimental import pallas as pl
from jax.experimental.pallas import tpu as pltpu

B_TILE = 256


def _zdec_kernel(phi_ref, x0_ref, x1_ref, w1phiT_ref, e0_ref, e1_ref,
                 dcat_ref, w2Tb_ref, b2c_ref, w3Tb_ref, b3c_ref, out_ref):
    j = pl.program_id(1)

    phiW = jnp.dot(phi_ref[...], w1phiT_ref[...],
                   preferred_element_type=jnp.float32)
    phiWcat = jnp.concatenate([phiW, phiW], axis=-1)

    x0col = jnp.transpose(x0_ref[...])
    s1 = x1_ref[0, j]
    cc = (x0col * e0_ref[...] + s1 * e1_ref[...] + dcat_ref[...])

    pre = phiWcat[:, None, :] + cc[None, :, :]
    h1 = jnp.maximum(pre, 0.0).reshape(B_TILE * 32, 128)
    h2 = jnp.maximum(
        jnp.dot(h1, w2Tb_ref[...], preferred_element_type=jnp.float32)
        + b2c_ref[...], 0.0)
    o = (jnp.dot(h2, w3Tb_ref[...], preferred_element_type=jnp.float32)
         + b3c_ref[...])
    out_ref[...] = o.reshape(B_TILE, 32, 32)


def kernel(phi, region_params, W1, b1, W2, b2, W3, b3):
    B, PHI = phi.shape
    levels, R, _ = region_params.shape
    H = W2.shape[0]
    O = W3.shape[0]
    K = R ** levels

    x0 = region_params[0, :, 0].reshape(1, R)
    x1 = region_params[1, :, 0].reshape(1, R)
    w1phiT = W1[:, 2:2 + PHI].T
    e0 = jnp.concatenate([W1[:, 0], W1[:, 1]]).reshape(1, 2 * H)
    e1 = jnp.concatenate([W1[:, 1], W1[:, 0]]).reshape(1, 2 * H)
    dcat = jnp.concatenate([b1 + W1[:, 2 + PHI],
                            b1 + W1[:, 3 + PHI]]).reshape(1, 2 * H)
    Z2 = jnp.zeros((H, H), W2.dtype)
    w2Tb = jnp.block([[W2.T, Z2], [Z2, W2.T]])
    b2c = jnp.concatenate([b2, b2]).reshape(1, 2 * H)
    Z3 = jnp.zeros((H, O), W3.dtype)
    w3Tb = jnp.block([[W3.T, Z3], [Z3, W3.T]])
    b3c = jnp.concatenate([b3, b3]).reshape(1, 2 * O)

    grid = (B // B_TILE, R)
    out = pl.pallas_call(
        _zdec_kernel,
        grid=grid,
        in_specs=[
            pl.BlockSpec((B_TILE, PHI), lambda i, j: (i, 0)),
            pl.BlockSpec((1, R), lambda i, j: (0, 0)),
            pl.BlockSpec(memory_space=pltpu.SMEM),
            pl.BlockSpec((PHI, H), lambda i, j: (0, 0)),
            pl.BlockSpec((1, 2 * H), lambda i, j: (0, 0)),
            pl.BlockSpec((1, 2 * H), lambda i, j: (0, 0)),
            pl.BlockSpec((1, 2 * H), lambda i, j: (0, 0)),
            pl.BlockSpec((2 * H, 2 * H), lambda i, j: (0, 0)),
            pl.BlockSpec((1, 2 * H), lambda i, j: (0, 0)),
            pl.BlockSpec((2 * H, 2 * O), lambda i, j: (0, 0)),
            pl.BlockSpec((1, 2 * O), lambda i, j: (0, 0)),
        ],
        out_specs=pl.BlockSpec((B_TILE, R, 2 * O), lambda i, j: (i, j, 0)),
        out_shape=jax.ShapeDtypeStruct((B, K, 2 * O), jnp.float32),
        compiler_params=pltpu.CompilerParams(
            dimension_semantics=("parallel", "parallel")),
        interpret=False,
    )(phi, x0, x1, w1phiT, e0, e1, dcat, w2Tb, b2c, w3Tb, b3c)
    return out
